# Initial kernel scaffold; baseline (speedup 1.0000x reference)
import jax
import jax.numpy as jnp
from jax.experimental import pallas as pl


def kernel(x, emb, w_lin, b_lin, W1, b1, W2, b2, W3, b3):
    raise NotImplementedError("write your pallas kernel here")


# speedup vs baseline: 27.7400x; 27.7400x over previous
import jax
import jax.numpy as jnp
import numpy as np
from jax import lax
from jax.experimental import pallas as pl
from jax.experimental.pallas import tpu as pltpu
from jax.experimental.pallas import tpu_sc as plsc

NUM_FIELDS = 26
EMBED_DIM = 16
FIELD_SIZE = 1000
FEATURE_DIM = NUM_FIELDS * FIELD_SIZE
BATCH = 4096
PAIRS = [(f, g) for f in range(NUM_FIELDS - 1) for g in range(f + 1, NUM_FIELDS)]
INTER_DIM = EMBED_DIM * len(PAIRS)
INTER_EXT = INTER_DIM + EMBED_DIM
_OFFS = np.arange(NUM_FIELDS, dtype=np.int32) * FIELD_SIZE

NUM_SC = 2
NUM_SUBCORES = 16
NUM_WORKERS = NUM_SC * NUM_SUBCORES
SAMPLES_PER_WORKER = BATCH // NUM_WORKERS


def _sc_body(xoff_hbm, emb_hbm, wpad_hbm, h_hbm, xoff_v, r_v, rw_v, hrow_v, sem):
    wid = lax.axis_index("s") * NUM_SC + lax.axis_index("c")
    base = wid * SAMPLES_PER_WORKER
    pltpu.sync_copy(xoff_hbm.at[pl.ds(base, SAMPLES_PER_WORKER)], xoff_v)

    def sample_body(s, carry):
        idx_row = xoff_v.at[s]
        descs = [
            pltpu.async_copy(emb_hbm.at[f].at[idx_row], r_v.at[f], sem)
            for f in range(NUM_FIELDS)
        ]
        dw = pltpu.async_copy(wpad_hbm.at[idx_row], rw_v, sem)
        for d in descs:
            d.wait()
        dw.wait()
        for p, (f, g) in enumerate(PAIRS):
            hrow_v[pl.ds(p * EMBED_DIM, EMBED_DIM)] = r_v[f, g, :] * r_v[g, f, :]
        acc = rw_v[0, :]
        for j in range(1, NUM_FIELDS):
            acc = acc + rw_v[j, :]
        hrow_v[pl.ds(INTER_DIM, EMBED_DIM)] = acc
        pltpu.sync_copy(hrow_v, h_hbm.at[base + s])
        return carry

    lax.fori_loop(0, SAMPLES_PER_WORKER, sample_body, 0)


def _sc_interactions(x_off, emb, w_pad):
    mesh = plsc.VectorSubcoreMesh(
        core_axis_name="c", subcore_axis_name="s",
        num_cores=NUM_SC, num_subcores=NUM_SUBCORES)
    return pl.kernel(
        _sc_body,
        out_type=jax.ShapeDtypeStruct((BATCH, INTER_EXT), jnp.float32),
        mesh=mesh,
        compiler_params=pltpu.CompilerParams(use_tc_tiling_on_sc=False),
        scratch_types=[
            pltpu.VMEM((SAMPLES_PER_WORKER, NUM_FIELDS), jnp.int32),
            pltpu.VMEM((NUM_FIELDS, NUM_FIELDS, EMBED_DIM), jnp.float32),
            pltpu.VMEM((NUM_FIELDS, EMBED_DIM), jnp.float32),
            pltpu.VMEM((INTER_EXT,), jnp.float32),
            pltpu.SemaphoreType.DMA,
        ],
    )(x_off, emb, w_pad)


def _mlp_body(h_ref, w1_ref, b1_ref, w2_ref, b2_ref, w3_ref, b3_ref, e_ref,
              out_ref):
    h = h_ref[...]
    first = jnp.dot(h, e_ref[...], preferred_element_type=jnp.float32)
    a = jnp.maximum(jnp.dot(h, w1_ref[...], preferred_element_type=jnp.float32)
                    + b1_ref[...], 0.0)
    a = jnp.maximum(jnp.dot(a, w2_ref[...], preferred_element_type=jnp.float32)
                    + b2_ref[...], 0.0)
    out = jnp.dot(a, w3_ref[...], preferred_element_type=jnp.float32)
    out_ref[...] = out + b3_ref[...] + first


def _mlp(h, W1e, b1, W2, b2, W3, b3f, evec):
    bb = 512
    grid = (BATCH // bb,)
    return pl.pallas_call(
        _mlp_body,
        grid=grid,
        in_specs=[
            pl.BlockSpec((bb, INTER_EXT), lambda i: (i, 0)),
            pl.BlockSpec((INTER_EXT, 64), lambda i: (0, 0)),
            pl.BlockSpec((1, 64), lambda i: (0, 0)),
            pl.BlockSpec((64, 32), lambda i: (0, 0)),
            pl.BlockSpec((1, 32), lambda i: (0, 0)),
            pl.BlockSpec((32, 1), lambda i: (0, 0)),
            pl.BlockSpec((1, 1), lambda i: (0, 0)),
            pl.BlockSpec((INTER_EXT, 1), lambda i: (0, 0)),
        ],
        out_specs=pl.BlockSpec((bb, 1), lambda i: (i, 0)),
        out_shape=jax.ShapeDtypeStruct((BATCH, 1), jnp.float32),
    )(h, W1e, b1, W2, b2, W3, b3f, evec)


def kernel(x, emb, w_lin, b_lin, W1, b1, W2, b2, W3, b3):
    x_off = x + jnp.asarray(_OFFS)[None, :]
    w_pad = jnp.pad(w_lin, ((0, 0), (0, EMBED_DIM - 1)))
    h = _sc_interactions(x_off, emb, w_pad)
    W1e = jnp.pad(W1, ((0, EMBED_DIM), (0, 0)))
    evec = jnp.zeros((INTER_EXT, 1), jnp.float32).at[INTER_DIM, 0].set(1.0)
    b3f = (b3 + b_lin).reshape(1, 1)
    out = _mlp(h, W1e, b1.reshape(1, 64), W2, b2.reshape(1, 32), W3, b3f, evec)
    return out[:, 0]


# speedup vs baseline: 37.1365x; 1.3387x over previous
import jax
import jax.numpy as jnp
import numpy as np
from jax import lax
from jax.experimental import pallas as pl
from jax.experimental.pallas import tpu as pltpu
from jax.experimental.pallas import tpu_sc as plsc

NUM_FIELDS = 26
EMBED_DIM = 16
FIELD_SIZE = 1000
BATCH = 4096
PAIRS = [(f, g) for f in range(NUM_FIELDS - 1) for g in range(f + 1, NUM_FIELDS)]
INTER_DIM = EMBED_DIM * len(PAIRS)
INTER_EXT = INTER_DIM + EMBED_DIM
_OFFS = np.arange(NUM_FIELDS, dtype=np.int32) * FIELD_SIZE

SLAB = 512
TROWS = 26112
W_COL = NUM_FIELDS * EMBED_DIM

NUM_SC = 2
NUM_SUBCORES = 16
NUM_WORKERS = NUM_SC * NUM_SUBCORES
SAMPLES_PER_WORKER = BATCH // NUM_WORKERS


def _compact_body(emb_ref, w_ref, out_ref):
    x = emb_ref[...].reshape(NUM_FIELDS * EMBED_DIM, SLAB)
    xt = jnp.swapaxes(x, 0, 1)
    w = w_ref[...]
    z = jnp.zeros((SLAB, SLAB - W_COL - 1), jnp.float32)
    out_ref[...] = jnp.concatenate([xt, w, z], axis=1)


def _compact(emb_bt, w_lin):
    grid = (TROWS // SLAB,)
    return pl.pallas_call(
        _compact_body,
        grid=grid,
        in_specs=[
            pl.BlockSpec((NUM_FIELDS, EMBED_DIM, SLAB), lambda i: (0, 0, i)),
            pl.BlockSpec((SLAB, 1), lambda i: (i, 0)),
        ],
        out_specs=pl.BlockSpec((SLAB, SLAB), lambda i: (i, 0)),
        out_shape=jax.ShapeDtypeStruct((TROWS, SLAB), jnp.float32),
    )(emb_bt, w_lin)


def _sc_body(xoff_hbm, tab_hbm, h_hbm, xoff_v, slab_v, hrow_v, sem):
    wid = lax.axis_index("s") * NUM_SC + lax.axis_index("c")
    base = wid * SAMPLES_PER_WORKER
    pltpu.sync_copy(xoff_hbm.at[pl.ds(base, SAMPLES_PER_WORKER)], xoff_v)

    def sample_body(s, carry):
        idx_row = xoff_v.at[s]
        pltpu.async_copy(tab_hbm.at[idx_row], slab_v, sem).wait()
        for p, (f, g) in enumerate(PAIRS):
            hrow_v[pl.ds(p * EMBED_DIM, EMBED_DIM)] = (
                slab_v[g, pl.ds(f * EMBED_DIM, EMBED_DIM)]
                * slab_v[f, pl.ds(g * EMBED_DIM, EMBED_DIM)])
        acc = slab_v[0, pl.ds(W_COL, EMBED_DIM)]
        for j in range(1, NUM_FIELDS):
            acc = acc + slab_v[j, pl.ds(W_COL, EMBED_DIM)]
        hrow_v[pl.ds(INTER_DIM, EMBED_DIM)] = acc
        pltpu.sync_copy(hrow_v, h_hbm.at[base + s])
        return carry

    lax.fori_loop(0, SAMPLES_PER_WORKER, sample_body, 0)


def _sc_interactions(x_off, tab):
    mesh = plsc.VectorSubcoreMesh(
        core_axis_name="c", subcore_axis_name="s",
        num_cores=NUM_SC, num_subcores=NUM_SUBCORES)
    return pl.kernel(
        _sc_body,
        out_type=jax.ShapeDtypeStruct((BATCH, INTER_EXT), jnp.float32),
        mesh=mesh,
        compiler_params=pltpu.CompilerParams(use_tc_tiling_on_sc=False),
        scratch_types=[
            pltpu.VMEM((SAMPLES_PER_WORKER, NUM_FIELDS), jnp.int32),
            pltpu.VMEM((NUM_FIELDS, SLAB), jnp.float32),
            pltpu.VMEM((INTER_EXT,), jnp.float32),
            pltpu.SemaphoreType.DMA,
        ],
    )(x_off, tab)


def _mlp_body(h_ref, w1_ref, b1_ref, w2_ref, b2_ref, w3_ref, b3_ref, e_ref,
              out_ref):
    h = h_ref[...]
    first = jnp.dot(h, e_ref[...], preferred_element_type=jnp.float32)
    a = jnp.maximum(jnp.dot(h, w1_ref[...], preferred_element_type=jnp.float32)
                    + b1_ref[...], 0.0)
    a = jnp.maximum(jnp.dot(a, w2_ref[...], preferred_element_type=jnp.float32)
                    + b2_ref[...], 0.0)
    out = jnp.dot(a, w3_ref[...], preferred_element_type=jnp.float32)
    out_ref[...] = out + b3_ref[...] + first


def _mlp(h, W1e, b1, W2, b2, W3, b3f, evec):
    bb = 512
    grid = (BATCH // bb,)
    return pl.pallas_call(
        _mlp_body,
        grid=grid,
        in_specs=[
            pl.BlockSpec((bb, INTER_EXT), lambda i: (i, 0)),
            pl.BlockSpec((INTER_EXT, 64), lambda i: (0, 0)),
            pl.BlockSpec((1, 64), lambda i: (0, 0)),
            pl.BlockSpec((64, 32), lambda i: (0, 0)),
            pl.BlockSpec((1, 32), lambda i: (0, 0)),
            pl.BlockSpec((32, 1), lambda i: (0, 0)),
            pl.BlockSpec((1, 1), lambda i: (0, 0)),
            pl.BlockSpec((INTER_EXT, 1), lambda i: (0, 0)),
        ],
        out_specs=pl.BlockSpec((bb, 1), lambda i: (i, 0)),
        out_shape=jax.ShapeDtypeStruct((BATCH, 1), jnp.float32),
    )(h, W1e, b1, W2, b2, W3, b3f, evec)


def kernel(x, emb, w_lin, b_lin, W1, b1, W2, b2, W3, b3):
    x_off = x + jnp.asarray(_OFFS)[None, :]
    emb_bt = jnp.transpose(emb, (0, 2, 1))
    tab = _compact(emb_bt, w_lin)
    h = _sc_interactions(x_off, tab)
    W1e = jnp.pad(W1, ((0, EMBED_DIM), (0, 0)))
    evec = jnp.zeros((INTER_EXT, 1), jnp.float32).at[INTER_DIM, 0].set(1.0)
    b3f = (b3 + b_lin).reshape(1, 1)
    out = _mlp(h, W1e, b1.reshape(1, 64), W2, b2.reshape(1, 32), W3, b3f, evec)
    return out[:, 0]


# speedup vs baseline: 55.8894x; 1.5050x over previous
import jax
import jax.numpy as jnp
import numpy as np
from jax import lax
from jax.experimental import pallas as pl
from jax.experimental.pallas import tpu as pltpu
from jax.experimental.pallas import tpu_sc as plsc

NUM_FIELDS = 26
EMBED_DIM = 16
FIELD_SIZE = 1000
BATCH = 4096
PAIRS = [(f, g) for f in range(NUM_FIELDS - 1) for g in range(f + 1, NUM_FIELDS)]
INTER_DIM = EMBED_DIM * len(PAIRS)
_OFFS = np.arange(NUM_FIELDS, dtype=np.int32) * FIELD_SIZE

SLAB = 512
TROWS = 26112
W_COL = NUM_FIELDS * EMBED_DIM

NTILE = 41
HCOLS = NTILE * 128

NUM_SC = 2
NUM_SUBCORES = 16
NUM_WORKERS = NUM_SC * NUM_SUBCORES
SAMPLES_PER_WORKER = BATCH // NUM_WORKERS
BLOCKS_PER_WORKER = SAMPLES_PER_WORKER // 8


def _compact_body(emb_ref, w_ref, out_ref):
    x = emb_ref[...].reshape(NUM_FIELDS * EMBED_DIM, SLAB)
    xt = jnp.swapaxes(x, 0, 1)
    w = w_ref[...]
    z = jnp.zeros((SLAB, SLAB - W_COL - 1), jnp.float32)
    out_ref[...] = jnp.concatenate([xt, w, z], axis=1)


def _compact(emb_bt, w_lin):
    grid = (TROWS // SLAB,)
    return pl.pallas_call(
        _compact_body,
        grid=grid,
        in_specs=[
            pl.BlockSpec((NUM_FIELDS, EMBED_DIM, SLAB), lambda i: (0, 0, i)),
            pl.BlockSpec((SLAB, 1), lambda i: (i, 0)),
        ],
        out_specs=pl.BlockSpec((SLAB, SLAB), lambda i: (i, 0)),
        out_shape=jax.ShapeDtypeStruct((TROWS, SLAB), jnp.float32),
    )(emb_bt, w_lin)


def _sc_body(xoff_hbm, tab_hbm, h4_hbm, xoff_v, slab0_v, slab1_v, hrow0_v,
             hrow1_v, gsem0, gsem1, hsem0, hsem1):
    wid = lax.axis_index("s") * NUM_SC + lax.axis_index("c")
    base = wid * SAMPLES_PER_WORKER
    rowblk0 = wid * BLOCKS_PER_WORKER
    pltpu.sync_copy(xoff_hbm.at[pl.ds(base, SAMPLES_PER_WORKER)], xoff_v)

    zero16 = jnp.zeros((EMBED_DIM,), jnp.float32)
    for hrow in (hrow0_v, hrow1_v):
        hrow[NTILE - 1, pl.ds(96, EMBED_DIM)] = zero16
        hrow[NTILE - 1, pl.ds(112, EMBED_DIM)] = zero16

    def products(cs, hrow):
        for p, (f, g) in enumerate(PAIRS):
            hrow[p // 8, pl.ds(EMBED_DIM * (p % 8), EMBED_DIM)] = (
                cs[g, pl.ds(f * EMBED_DIM, EMBED_DIM)]
                * cs[f, pl.ds(g * EMBED_DIM, EMBED_DIM)])
        acc = cs[0, pl.ds(W_COL, EMBED_DIM)]
        for i in range(1, NUM_FIELDS):
            acc = acc + cs[i, pl.ds(W_COL, EMBED_DIM)]
        hrow[NTILE - 1, pl.ds(80, EMBED_DIM)] = acc

    def hout(s):
        return h4_hbm.at[rowblk0 + s // 8, :, s % 8]

    pltpu.async_copy(tab_hbm.at[xoff_v.at[0]], slab0_v, gsem0)

    def pair_body(t, carry):
        s0 = 2 * t
        @pl.when(t >= 1)
        def _():
            pltpu.make_async_copy(hout(0), hrow0_v, hsem0).wait()
        pltpu.make_async_copy(tab_hbm.at[pl.ds(0, NUM_FIELDS)], slab0_v,
                              gsem0).wait()
        pltpu.async_copy(tab_hbm.at[xoff_v.at[s0 + 1]], slab1_v, gsem1)
        products(slab0_v, hrow0_v)
        pltpu.async_copy(hrow0_v, hout(s0), hsem0)
        @pl.when(t >= 1)
        def _():
            pltpu.make_async_copy(hout(0), hrow1_v, hsem1).wait()
        pltpu.make_async_copy(tab_hbm.at[pl.ds(0, NUM_FIELDS)], slab1_v,
                              gsem1).wait()
        snxt = jnp.minimum(s0 + 2, SAMPLES_PER_WORKER - 1)
        pltpu.async_copy(tab_hbm.at[xoff_v.at[snxt]], slab0_v, gsem0)
        products(slab1_v, hrow1_v)
        pltpu.async_copy(hrow1_v, hout(s0 + 1), hsem1)
        return carry

    lax.fori_loop(0, SAMPLES_PER_WORKER // 2, pair_body, 0)
    pltpu.make_async_copy(hout(0), hrow0_v, hsem0).wait()
    pltpu.make_async_copy(hout(0), hrow1_v, hsem1).wait()
    pltpu.make_async_copy(tab_hbm.at[pl.ds(0, NUM_FIELDS)], slab0_v,
                          gsem0).wait()


def _sc_interactions(x_off, tab):
    mesh = plsc.VectorSubcoreMesh(
        core_axis_name="c", subcore_axis_name="s",
        num_cores=NUM_SC, num_subcores=NUM_SUBCORES)
    return pl.kernel(
        _sc_body,
        out_type=jax.ShapeDtypeStruct((BATCH // 8, NTILE, 8, 128), jnp.float32),
        mesh=mesh,
        compiler_params=pltpu.CompilerParams(use_tc_tiling_on_sc=False),
        scratch_types=[
            pltpu.VMEM((SAMPLES_PER_WORKER, NUM_FIELDS), jnp.int32),
            pltpu.VMEM((NUM_FIELDS, SLAB), jnp.float32),
            pltpu.VMEM((NUM_FIELDS, SLAB), jnp.float32),
            pltpu.VMEM((NTILE, 128), jnp.float32),
            pltpu.VMEM((NTILE, 128), jnp.float32),
            pltpu.SemaphoreType.DMA,
            pltpu.SemaphoreType.DMA,
            pltpu.SemaphoreType.DMA,
            pltpu.SemaphoreType.DMA,
        ],
    )(x_off, tab)


def _mlp_body(h_ref, w1_ref, b1_ref, w2_ref, b2_ref, w3_ref, b3_ref, e_ref,
              out_ref):
    x = h_ref[...]
    acc = jnp.broadcast_to(b1_ref[...], (512, 64))
    for c in range(NTILE):
        piece = x[:, c, :, :].reshape(512, 128)
        acc = acc + jnp.dot(piece, w1_ref[c],
                            preferred_element_type=jnp.float32)
    last = x[:, NTILE - 1, :, :].reshape(512, 128)
    first = jnp.dot(last, e_ref[...], preferred_element_type=jnp.float32)
    a = jnp.maximum(acc, 0.0)
    a = jnp.maximum(jnp.dot(a, w2_ref[...], preferred_element_type=jnp.float32)
                    + b2_ref[...], 0.0)
    out = jnp.dot(a, w3_ref[...], preferred_element_type=jnp.float32)
    out_ref[...] = out + b3_ref[...] + first


def _mlp(h4, W1r, b1, W2, b2, W3, b3f, e128):
    grid = (BATCH // 512,)
    return pl.pallas_call(
        _mlp_body,
        grid=grid,
        in_specs=[
            pl.BlockSpec((64, NTILE, 8, 128), lambda i: (i, 0, 0, 0)),
            pl.BlockSpec((NTILE, 128, 64), lambda i: (0, 0, 0)),
            pl.BlockSpec((1, 64), lambda i: (0, 0)),
            pl.BlockSpec((64, 32), lambda i: (0, 0)),
            pl.BlockSpec((1, 32), lambda i: (0, 0)),
            pl.BlockSpec((32, 1), lambda i: (0, 0)),
            pl.BlockSpec((1, 1), lambda i: (0, 0)),
            pl.BlockSpec((128, 1), lambda i: (0, 0)),
        ],
        out_specs=pl.BlockSpec((512, 1), lambda i: (i, 0)),
        out_shape=jax.ShapeDtypeStruct((BATCH, 1), jnp.float32),
    )(h4, W1r, b1, W2, b2, W3, b3f, e128)


def kernel(x, emb, w_lin, b_lin, W1, b1, W2, b2, W3, b3):
    x_off = x + jnp.asarray(_OFFS)[None, :]
    emb_bt = jnp.transpose(emb, (0, 2, 1))
    tab = _compact(emb_bt, w_lin)
    h4 = _sc_interactions(x_off, tab)
    W1r = jnp.pad(W1, ((0, HCOLS - INTER_DIM), (0, 0))).reshape(NTILE, 128, 64)
    e128 = jnp.zeros((128, 1), jnp.float32).at[80, 0].set(1.0)
    b3f = (b3 + b_lin).reshape(1, 1)
    out = _mlp(h4, W1r, b1.reshape(1, 64), W2, b2.reshape(1, 32), W3, b3f,
               e128)
    return out[:, 0]
